# baseline (speedup 1.0000x reference)
import functools

import jax
import jax.numpy as jnp
from jax import lax
from jax.experimental import pallas as pl
from jax.experimental.pallas import tpu as pltpu
from jax.experimental.pallas import tpu_sc as plsc

HEADS = 8
DIM_HEAD = 64
INNER = HEADS * DIM_HEAD
REPS = 64
ROW = 128
QDIM = 128
SCALE = DIM_HEAD ** (-0.5)
K_TOP = 8
BM = 8192
INT_MAX = jnp.iinfo(jnp.int32).max


def _scan_body(m_real, x_ref, wq_ref, wqe_ref, ctx_ref, q_out, d2_out, e_s):
    b = pl.program_id(0)
    i = pl.program_id(1)
    _, bm, _ = ctx_ref.shape
    B = e_s.shape[0]
    n = x_ref.shape[0] // B

    @pl.when(jnp.logical_and(b == 0, i == 0))
    def _init():
        q = x_ref[...] @ wq_ref[...]
        q_out[...] = q
        xs = jnp.concatenate([x_ref[k * n:k * n + 1, :] for k in range(B)], axis=0)
        e_s[...] = (xs @ wq_ref[...]) @ wqe_ref[...]

    c = ctx_ref[...][:, :, :REPS]
    e = e_s[pl.ds(b, 1), :]
    diff = c - e[:, None, :]
    d2 = jnp.sum(diff * diff, axis=-1)
    col = lax.broadcasted_iota(jnp.int32, (1, bm), 1) + i * bm
    d2m = jnp.where(col < m_real, d2, jnp.inf)
    d2_out[...] = d2m.reshape(1, 1, 8, bm // 8)


def _top8_body(m_real, d2_ref, idx_out):
    B, mp = d2_ref.shape
    vals = d2_ref[...]
    gidx = lax.broadcasted_iota(jnp.int32, (B, mp), 1)
    ni = []
    for _ in range(K_TOP):
        m = jnp.min(vals, axis=1, keepdims=True)
        g = jnp.min(jnp.where(vals <= m, gidx, INT_MAX), axis=1, keepdims=True)
        ni.append(g)
        vals = jnp.where(gidx == g, jnp.inf, vals)
    boff = lax.broadcasted_iota(jnp.int32, (B, K_TOP), 0) * m_real
    idx_out[...] = jnp.concatenate(ni, axis=1) + boff


def _scan_call(x2, Wq, Wqe, context):
    B, M = context.shape[0], context.shape[1]
    n = x2.shape[0] // B
    nb = -(-M // BM)
    mp = nb * BM
    return pl.pallas_call(
        functools.partial(_scan_body, M),
        grid=(B, nb),
        in_specs=[
            pl.BlockSpec(x2.shape, lambda b, i: (0, 0)),
            pl.BlockSpec(Wq.shape, lambda b, i: (0, 0)),
            pl.BlockSpec(Wqe.shape, lambda b, i: (0, 0)),
            pl.BlockSpec((1, BM, ROW), lambda b, i: (b, i, 0)),
        ],
        out_specs=[
            pl.BlockSpec((B * n, INNER), lambda b, i: (0, 0)),
            pl.BlockSpec((1, 1, 8, BM // 8), lambda b, i: (b, i, 0, 0)),
        ],
        out_shape=[
            jax.ShapeDtypeStruct((B * n, INNER), jnp.float32),
            jax.ShapeDtypeStruct((B, nb, 8, BM // 8), jnp.float32),
        ],
        scratch_shapes=[
            pltpu.VMEM((B, REPS), jnp.float32),
        ],
    )(x2, Wq, Wqe, context)


def _top8_call(d2, m_real):
    B = d2.shape[0]
    return pl.pallas_call(
        functools.partial(_top8_body, m_real),
        out_shape=jax.ShapeDtypeStruct((B, K_TOP), jnp.int32),
    )(d2)


def _sc_gather(ctx2d, idx_flat):
    nrows = idx_flat.shape[0]
    mesh = plsc.VectorSubcoreMesh(core_axis_name="c", subcore_axis_name="s")

    @functools.partial(
        pl.kernel,
        out_type=jax.ShapeDtypeStruct((nrows, ROW), jnp.float32),
        mesh=mesh,
        scratch_types=[
            pltpu.VMEM((nrows,), jnp.int32),
            pltpu.VMEM((nrows, ROW), jnp.float32),
            pltpu.SemaphoreType.DMA,
        ],
    )
    def gather_kernel(ctx_hbm, idx_hbm, out_hbm, idx_v, rows_v, sem):
        cid = lax.axis_index("c")
        sid = lax.axis_index("s")

        @pl.when(jnp.logical_and(cid == 0, sid == 0))
        def _():
            pltpu.sync_copy(idx_hbm, idx_v)
            pltpu.async_copy(ctx_hbm.at[idx_v], rows_v, sem).wait()
            pltpu.sync_copy(rows_v, out_hbm)

    return gather_kernel(ctx2d, idx_flat)


def _attn_body(q_ref, sel_ref, wk_ref, wv_ref, wout_ref, bout_ref, o_ref):
    wk = wk_ref[...]
    wv = wv_ref[...]
    wout = wout_ref[...]
    bout = bout_ref[...]
    B = sel_ref.shape[0] // K_TOP
    n = q_ref.shape[0] // B
    for b in range(B):
        cs = sel_ref[b * K_TOP:(b + 1) * K_TOP, :]
        kb = cs[:, REPS:] @ wk
        vb = cs[:, :REPS] @ wv
        qb = q_ref[b * n:(b + 1) * n, :]
        outs = []
        for h in range(HEADS):
            sl = slice(h * DIM_HEAD, (h + 1) * DIM_HEAD)
            qh = qb[:, sl]
            kh = kb[:, sl]
            vh = vb[:, sl]
            sim = lax.dot_general(qh, kh, (((1,), (1,)), ((), ()))) * SCALE
            mmax = jnp.max(sim, axis=-1, keepdims=True)
            ex = jnp.exp(sim - mmax)
            attn = ex / jnp.sum(ex, axis=-1, keepdims=True)
            outs.append(attn @ vh)
        ob = jnp.concatenate(outs, axis=1)
        o_ref[b * n:(b + 1) * n, :] = ob @ wout + bout


def _attn_call(q2, sel, Wk, Wv, Wout, bout2):
    rows = q2.shape[0]
    return pl.pallas_call(
        _attn_body,
        out_shape=jax.ShapeDtypeStruct((rows, QDIM), jnp.float32),
    )(q2, sel, Wk, Wv, Wout, bout2)


def kernel(x, context, topk, Wq, Wk, Wv, Wqe, Wout, bout):
    B, N, _ = x.shape
    M = context.shape[1]
    x2 = x.reshape(B * N, QDIM)
    q2, d2 = _scan_call(x2, Wq, Wqe, context)
    idx = _top8_call(d2.reshape(B, -1), M)
    sel = _sc_gather(context.reshape(B * M, ROW), idx.reshape(B * K_TOP))
    out = _attn_call(q2, sel, Wk, Wv, Wout, bout.reshape(1, QDIM))
    return out.reshape(B, N, QDIM)


# speedup vs baseline: 1.2220x; 1.2220x over previous
import functools

import jax
import jax.numpy as jnp
from jax import lax
from jax.experimental import pallas as pl
from jax.experimental.pallas import tpu as pltpu
from jax.experimental.pallas import tpu_sc as plsc

HEADS = 8
DIM_HEAD = 64
INNER = HEADS * DIM_HEAD
REPS = 64
ROW = 128
QDIM = 128
SCALE = DIM_HEAD ** (-0.5)
K_TOP = 8
BM = 8192
INT_MAX = jnp.iinfo(jnp.int32).max


def _scan_body(m_real, x_ref, wq_ref, wqe_ref, ctx_ref, q_out, idx_out, e_s, d2_s):
    i = pl.program_id(0)
    nb = pl.num_programs(0)
    B, bm, _ = ctx_ref.shape
    n = x_ref.shape[0] // B
    m_pad = bm * nb

    @pl.when(i == 0)
    def _init():
        q = x_ref[...] @ wq_ref[...]
        q_out[...] = q
        xs = jnp.concatenate([x_ref[k * n:k * n + 1, :] for k in range(B)], axis=0)
        e_s[...] = (xs @ wq_ref[...]) @ wqe_ref[...]

    c = ctx_ref[...][:, :, :REPS]
    e = e_s[...]
    diff = c - e[:, None, :]
    d2 = jnp.sum(diff * diff, axis=-1)
    col = lax.broadcasted_iota(jnp.int32, (B, bm), 1) + i * bm
    d2_s[:, pl.ds(i * bm, bm)] = jnp.where(col < m_real, d2, jnp.inf)

    @pl.when(i == nb - 1)
    def _top8():
        vals = d2_s[...]
        gidx = lax.broadcasted_iota(jnp.int32, (B, m_pad), 1)
        ni = []
        for _ in range(K_TOP):
            m = jnp.min(vals, axis=1, keepdims=True)
            g = jnp.min(jnp.where(vals <= m, gidx, INT_MAX), axis=1, keepdims=True)
            ni.append(g)
            vals = jnp.where(gidx == g, jnp.inf, vals)
        boff = lax.broadcasted_iota(jnp.int32, (B, K_TOP), 0) * m_real
        idx_out[...] = jnp.concatenate(ni, axis=1) + boff


def _scan_call(x2, Wq, Wqe, context):
    B, M = context.shape[0], context.shape[1]
    n = x2.shape[0] // B
    nb = -(-M // BM)
    return pl.pallas_call(
        functools.partial(_scan_body, M),
        grid=(nb,),
        in_specs=[
            pl.BlockSpec(x2.shape, lambda i: (0, 0)),
            pl.BlockSpec(Wq.shape, lambda i: (0, 0)),
            pl.BlockSpec(Wqe.shape, lambda i: (0, 0)),
            pl.BlockSpec((B, BM, ROW), lambda i: (0, i, 0)),
        ],
        out_specs=[
            pl.BlockSpec((B * n, INNER), lambda i: (0, 0)),
            pl.BlockSpec((B, K_TOP), lambda i: (0, 0)),
        ],
        out_shape=[
            jax.ShapeDtypeStruct((B * n, INNER), jnp.float32),
            jax.ShapeDtypeStruct((B, K_TOP), jnp.int32),
        ],
        scratch_shapes=[
            pltpu.VMEM((B, REPS), jnp.float32),
            pltpu.VMEM((B, nb * BM), jnp.float32),
        ],
    )(x2, Wq, Wqe, context)


def _sc_gather(ctx2d, idx_flat):
    nrows = idx_flat.shape[0]
    mesh = plsc.VectorSubcoreMesh(core_axis_name="c", subcore_axis_name="s")

    @functools.partial(
        pl.kernel,
        out_type=jax.ShapeDtypeStruct((nrows, ROW), jnp.float32),
        mesh=mesh,
        scratch_types=[
            pltpu.VMEM((nrows,), jnp.int32),
            pltpu.VMEM((nrows, ROW), jnp.float32),
            pltpu.SemaphoreType.DMA,
        ],
    )
    def gather_kernel(ctx_hbm, idx_hbm, out_hbm, idx_v, rows_v, sem):
        cid = lax.axis_index("c")
        sid = lax.axis_index("s")

        @pl.when(jnp.logical_and(cid == 0, sid == 0))
        def _():
            pltpu.sync_copy(idx_hbm, idx_v)
            pltpu.async_copy(ctx_hbm.at[idx_v], rows_v, sem).wait()
            pltpu.sync_copy(rows_v, out_hbm)

    return gather_kernel(ctx2d, idx_flat)


def _attn_body(q_ref, sel_ref, wk_ref, wv_ref, wout_ref, bout_ref, o_ref):
    wk = wk_ref[...]
    wv = wv_ref[...]
    wout = wout_ref[...]
    bout = bout_ref[...]
    B = sel_ref.shape[0] // K_TOP
    n = q_ref.shape[0] // B
    for b in range(B):
        cs = sel_ref[b * K_TOP:(b + 1) * K_TOP, :]
        kb = cs[:, REPS:] @ wk
        vb = cs[:, :REPS] @ wv
        qb = q_ref[b * n:(b + 1) * n, :]
        outs = []
        for h in range(HEADS):
            sl = slice(h * DIM_HEAD, (h + 1) * DIM_HEAD)
            qh = qb[:, sl]
            kh = kb[:, sl]
            vh = vb[:, sl]
            sim = lax.dot_general(qh, kh, (((1,), (1,)), ((), ()))) * SCALE
            mmax = jnp.max(sim, axis=-1, keepdims=True)
            ex = jnp.exp(sim - mmax)
            attn = ex / jnp.sum(ex, axis=-1, keepdims=True)
            outs.append(attn @ vh)
        ob = jnp.concatenate(outs, axis=1)
        o_ref[b * n:(b + 1) * n, :] = ob @ wout + bout


def _attn_call(q2, sel, Wk, Wv, Wout, bout2):
    rows = q2.shape[0]
    return pl.pallas_call(
        _attn_body,
        out_shape=jax.ShapeDtypeStruct((rows, QDIM), jnp.float32),
    )(q2, sel, Wk, Wv, Wout, bout2)


def kernel(x, context, topk, Wq, Wk, Wv, Wqe, Wout, bout):
    B, N, _ = x.shape
    M = context.shape[1]
    x2 = x.reshape(B * N, QDIM)
    q2, idx = _scan_call(x2, Wq, Wqe, context)
    sel = _sc_gather(context.reshape(B * M, ROW), idx.reshape(B * K_TOP))
    out = _attn_call(q2, sel, Wk, Wv, Wout, bout.reshape(1, QDIM))
    return out.reshape(B, N, QDIM)


# speedup vs baseline: 1.2598x; 1.0309x over previous
import functools

import jax
import jax.numpy as jnp
from jax import lax
from jax.experimental import pallas as pl
from jax.experimental.pallas import tpu as pltpu
from jax.experimental.pallas import tpu_sc as plsc

HEADS = 8
DIM_HEAD = 64
INNER = HEADS * DIM_HEAD
REPS = 64
ROW = 128
QDIM = 128
SCALE = DIM_HEAD ** (-0.5)
K_TOP = 8
BM = 6272
INT_MAX = jnp.iinfo(jnp.int32).max


def _scan_body(m_real, x_ref, wq_ref, wqe_ref, ctx_ref, q_out, idx_out, e_s, d2_s):
    i = pl.program_id(0)
    nb = pl.num_programs(0)
    B, bm, _ = ctx_ref.shape
    n = x_ref.shape[0] // B
    m_pad = bm * nb

    @pl.when(i == 0)
    def _init():
        q = x_ref[...] @ wq_ref[...]
        q_out[...] = q
        xs = jnp.concatenate([x_ref[k * n:k * n + 1, :] for k in range(B)], axis=0)
        e_s[...] = (xs @ wq_ref[...]) @ wqe_ref[...]

    c = ctx_ref[...][:, :, :REPS]
    e = e_s[...]
    diff = c - e[:, None, :]
    d2 = jnp.sum(diff * diff, axis=-1)
    col = lax.broadcasted_iota(jnp.int32, (B, bm), 1) + i * bm
    d2_s[:, pl.ds(i * bm, bm)] = jnp.where(col < m_real, d2, jnp.inf)

    @pl.when(i == nb - 1)
    def _top8():
        vals = d2_s[...]
        gidx = lax.broadcasted_iota(jnp.int32, (B, m_pad), 1)
        ni = []
        for _ in range(K_TOP):
            m = jnp.min(vals, axis=1, keepdims=True)
            g = jnp.min(jnp.where(vals <= m, gidx, INT_MAX), axis=1, keepdims=True)
            ni.append(g)
            vals = jnp.where(gidx == g, jnp.inf, vals)
        boff = lax.broadcasted_iota(jnp.int32, (B, K_TOP), 0) * m_real
        idx_out[...] = jnp.concatenate(ni, axis=1) + boff


def _scan_call(x2, Wq, Wqe, context):
    B, M = context.shape[0], context.shape[1]
    n = x2.shape[0] // B
    nb = -(-M // BM)
    return pl.pallas_call(
        functools.partial(_scan_body, M),
        grid=(nb,),
        in_specs=[
            pl.BlockSpec(x2.shape, lambda i: (0, 0)),
            pl.BlockSpec(Wq.shape, lambda i: (0, 0)),
            pl.BlockSpec(Wqe.shape, lambda i: (0, 0)),
            pl.BlockSpec((B, BM, ROW), lambda i: (0, i, 0)),
        ],
        out_specs=[
            pl.BlockSpec((B * n, INNER), lambda i: (0, 0)),
            pl.BlockSpec((B, K_TOP), lambda i: (0, 0)),
        ],
        out_shape=[
            jax.ShapeDtypeStruct((B * n, INNER), jnp.float32),
            jax.ShapeDtypeStruct((B, K_TOP), jnp.int32),
        ],
        scratch_shapes=[
            pltpu.VMEM((B, REPS), jnp.float32),
            pltpu.VMEM((B, nb * BM), jnp.float32),
        ],
    )(x2, Wq, Wqe, context)


def _sc_gather(ctx2d, idx_flat):
    nrows = idx_flat.shape[0]
    mesh = plsc.VectorSubcoreMesh(core_axis_name="c", subcore_axis_name="s")

    @functools.partial(
        pl.kernel,
        out_type=jax.ShapeDtypeStruct((nrows, ROW), jnp.float32),
        mesh=mesh,
        scratch_types=[
            pltpu.VMEM((nrows,), jnp.int32),
            pltpu.VMEM((nrows, ROW), jnp.float32),
            pltpu.SemaphoreType.DMA,
        ],
    )
    def gather_kernel(ctx_hbm, idx_hbm, out_hbm, idx_v, rows_v, sem):
        cid = lax.axis_index("c")
        sid = lax.axis_index("s")

        @pl.when(jnp.logical_and(cid == 0, sid == 0))
        def _():
            pltpu.sync_copy(idx_hbm, idx_v)
            pltpu.async_copy(ctx_hbm.at[idx_v], rows_v, sem).wait()
            pltpu.sync_copy(rows_v, out_hbm)

    return gather_kernel(ctx2d, idx_flat)


def _attn_body(q_ref, sel_ref, wk_ref, wv_ref, wout_ref, bout_ref, o_ref):
    wk = wk_ref[...]
    wv = wv_ref[...]
    wout = wout_ref[...]
    bout = bout_ref[...]
    B = sel_ref.shape[0] // K_TOP
    n = q_ref.shape[0] // B
    for b in range(B):
        cs = sel_ref[b * K_TOP:(b + 1) * K_TOP, :]
        kb = cs[:, REPS:] @ wk
        vb = cs[:, :REPS] @ wv
        qb = q_ref[b * n:(b + 1) * n, :]
        outs = []
        for h in range(HEADS):
            sl = slice(h * DIM_HEAD, (h + 1) * DIM_HEAD)
            qh = qb[:, sl]
            kh = kb[:, sl]
            vh = vb[:, sl]
            sim = lax.dot_general(qh, kh, (((1,), (1,)), ((), ()))) * SCALE
            mmax = jnp.max(sim, axis=-1, keepdims=True)
            ex = jnp.exp(sim - mmax)
            attn = ex / jnp.sum(ex, axis=-1, keepdims=True)
            outs.append(attn @ vh)
        ob = jnp.concatenate(outs, axis=1)
        o_ref[b * n:(b + 1) * n, :] = ob @ wout + bout


def _attn_call(q2, sel, Wk, Wv, Wout, bout2):
    rows = q2.shape[0]
    return pl.pallas_call(
        _attn_body,
        out_shape=jax.ShapeDtypeStruct((rows, QDIM), jnp.float32),
    )(q2, sel, Wk, Wv, Wout, bout2)


def kernel(x, context, topk, Wq, Wk, Wv, Wqe, Wout, bout):
    B, N, _ = x.shape
    M = context.shape[1]
    x2 = x.reshape(B * N, QDIM)
    q2, idx = _scan_call(x2, Wq, Wqe, context)
    sel = _sc_gather(context.reshape(B * M, ROW), idx.reshape(B * K_TOP))
    out = _attn_call(q2, sel, Wk, Wv, Wout, bout.reshape(1, QDIM))
    return out.reshape(B, N, QDIM)


# speedup vs baseline: 1.2631x; 1.0027x over previous
import functools

import jax
import jax.numpy as jnp
from jax import lax
from jax.experimental import pallas as pl
from jax.experimental.pallas import tpu as pltpu
from jax.experimental.pallas import tpu_sc as plsc

HEADS = 8
DIM_HEAD = 64
INNER = HEADS * DIM_HEAD
REPS = 64
ROW = 128
QDIM = 128
SCALE = DIM_HEAD ** (-0.5)
K_TOP = 8
BM = 7168
INT_MAX = jnp.iinfo(jnp.int32).max


def _scan_body(m_real, x_ref, wq_ref, wqe_ref, ctx_ref, q_out, idx_out, e_s, d2_s):
    i = pl.program_id(0)
    nb = pl.num_programs(0)
    B, bm, _ = ctx_ref.shape
    n = x_ref.shape[0] // B
    m_pad = bm * nb

    @pl.when(i == 0)
    def _init():
        q = x_ref[...] @ wq_ref[...]
        q_out[...] = q
        xs = jnp.concatenate([x_ref[k * n:k * n + 1, :] for k in range(B)], axis=0)
        e_s[...] = (xs @ wq_ref[...]) @ wqe_ref[...]

    c = ctx_ref[...][:, :, :REPS]
    e = e_s[...]
    diff = c - e[:, None, :]
    d2 = jnp.sum(diff * diff, axis=-1)
    col = lax.broadcasted_iota(jnp.int32, (B, bm), 1) + i * bm
    d2_s[:, pl.ds(i * bm, bm)] = jnp.where(col < m_real, d2, jnp.inf)

    @pl.when(i == nb - 1)
    def _top8():
        vals = d2_s[...]
        gidx = lax.broadcasted_iota(jnp.int32, (B, m_pad), 1)
        ni = []
        for _ in range(K_TOP):
            m = jnp.min(vals, axis=1, keepdims=True)
            g = jnp.min(jnp.where(vals <= m, gidx, INT_MAX), axis=1, keepdims=True)
            ni.append(g)
            vals = jnp.where(gidx == g, jnp.inf, vals)
        boff = lax.broadcasted_iota(jnp.int32, (B, K_TOP), 0) * m_real
        idx_out[...] = jnp.concatenate(ni, axis=1) + boff


def _scan_call(x2, Wq, Wqe, context):
    B, M = context.shape[0], context.shape[1]
    n = x2.shape[0] // B
    nb = -(-M // BM)
    return pl.pallas_call(
        functools.partial(_scan_body, M),
        grid=(nb,),
        in_specs=[
            pl.BlockSpec(x2.shape, lambda i: (0, 0)),
            pl.BlockSpec(Wq.shape, lambda i: (0, 0)),
            pl.BlockSpec(Wqe.shape, lambda i: (0, 0)),
            pl.BlockSpec((B, BM, ROW), lambda i: (0, i, 0)),
        ],
        out_specs=[
            pl.BlockSpec((B * n, INNER), lambda i: (0, 0)),
            pl.BlockSpec((B, K_TOP), lambda i: (0, 0)),
        ],
        out_shape=[
            jax.ShapeDtypeStruct((B * n, INNER), jnp.float32),
            jax.ShapeDtypeStruct((B, K_TOP), jnp.int32),
        ],
        scratch_shapes=[
            pltpu.VMEM((B, REPS), jnp.float32),
            pltpu.VMEM((B, nb * BM), jnp.float32),
        ],
    )(x2, Wq, Wqe, context)


def _sc_gather(ctx2d, idx_flat):
    nrows = idx_flat.shape[0]
    mesh = plsc.VectorSubcoreMesh(core_axis_name="c", subcore_axis_name="s")

    @functools.partial(
        pl.kernel,
        out_type=jax.ShapeDtypeStruct((nrows, ROW), jnp.float32),
        mesh=mesh,
        scratch_types=[
            pltpu.VMEM((nrows,), jnp.int32),
            pltpu.VMEM((nrows, ROW), jnp.float32),
            pltpu.SemaphoreType.DMA,
        ],
    )
    def gather_kernel(ctx_hbm, idx_hbm, out_hbm, idx_v, rows_v, sem):
        cid = lax.axis_index("c")
        sid = lax.axis_index("s")

        @pl.when(jnp.logical_and(cid == 0, sid == 0))
        def _():
            pltpu.sync_copy(idx_hbm, idx_v)
            pltpu.async_copy(ctx_hbm.at[idx_v], rows_v, sem).wait()
            pltpu.sync_copy(rows_v, out_hbm)

    return gather_kernel(ctx2d, idx_flat)


def _attn_body(q_ref, sel_ref, wk_ref, wv_ref, wout_ref, bout_ref, o_ref):
    wk = wk_ref[...]
    wv = wv_ref[...]
    wout = wout_ref[...]
    bout = bout_ref[...]
    B = sel_ref.shape[0] // K_TOP
    n = q_ref.shape[0] // B
    for b in range(B):
        cs = sel_ref[b * K_TOP:(b + 1) * K_TOP, :]
        kb = cs[:, REPS:] @ wk
        vb = cs[:, :REPS] @ wv
        qb = q_ref[b * n:(b + 1) * n, :]
        outs = []
        for h in range(HEADS):
            sl = slice(h * DIM_HEAD, (h + 1) * DIM_HEAD)
            qh = qb[:, sl]
            kh = kb[:, sl]
            vh = vb[:, sl]
            sim = lax.dot_general(qh, kh, (((1,), (1,)), ((), ()))) * SCALE
            mmax = jnp.max(sim, axis=-1, keepdims=True)
            ex = jnp.exp(sim - mmax)
            attn = ex / jnp.sum(ex, axis=-1, keepdims=True)
            outs.append(attn @ vh)
        ob = jnp.concatenate(outs, axis=1)
        o_ref[b * n:(b + 1) * n, :] = ob @ wout + bout


def _attn_call(q2, sel, Wk, Wv, Wout, bout2):
    rows = q2.shape[0]
    return pl.pallas_call(
        _attn_body,
        out_shape=jax.ShapeDtypeStruct((rows, QDIM), jnp.float32),
    )(q2, sel, Wk, Wv, Wout, bout2)


def kernel(x, context, topk, Wq, Wk, Wv, Wqe, Wout, bout):
    B, N, _ = x.shape
    M = context.shape[1]
    x2 = x.reshape(B * N, QDIM)
    q2, idx = _scan_call(x2, Wq, Wqe, context)
    sel = _sc_gather(context.reshape(B * M, ROW), idx.reshape(B * K_TOP))
    out = _attn_call(q2, sel, Wk, Wv, Wout, bout.reshape(1, QDIM))
    return out.reshape(B, N, QDIM)
